# baseline (speedup 1.0000x reference)
import functools

import jax
import jax.numpy as jnp
from jax import lax
from jax.experimental import pallas as pl
from jax.experimental.pallas import tpu as pltpu
from jax.experimental.pallas import tpu_sc as plsc

N = 10000
E = 320000
D = 128
H = 128
O_ = 128
EF = 16

NC = 2
NS = 16
NW = NC * NS
EPW = E // NW
CH = 80
NCH = EPW // CH
NP = 10240
RPT = NP // NS
AW = 144


def _silu(x):
    return x / (1.0 + jnp.exp(-x))


D2 = D // 2


def _pack_cols(x_lo, x_hi):
    lo = jax.lax.bitcast_convert_type(
        x_lo.astype(jnp.bfloat16), jnp.uint16).astype(jnp.uint32)
    hi = jax.lax.bitcast_convert_type(
        x_hi.astype(jnp.bfloat16), jnp.uint16).astype(jnp.uint32)
    return jax.lax.bitcast_convert_type((hi << 16) | lo, jnp.float32)


def _unpack_cols(xp):
    w = jax.lax.bitcast_convert_type(xp, jnp.uint32)
    lo = jax.lax.bitcast_convert_type(w << 16, jnp.float32)
    hi = jax.lax.bitcast_convert_type(w & jnp.uint32(0xFFFF0000), jnp.float32)
    return jnp.concatenate([lo, hi], axis=1)


def _pq_body(nf, wpq, bpq, p_out, q_out):
    pq = jnp.dot(nf[...], wpq[...], preferred_element_type=jnp.float32)
    pq = pq + bpq[...]
    p_out[...] = _pack_cols(pq[:, 0:D2], pq[:, D2:D])
    q_out[...] = _pack_cols(pq[:, D:D + D2], pq[:, D + D2:])


def _pq_call(nf, wpq, bpq):
    return pl.pallas_call(
        _pq_body,
        out_shape=(
            jax.ShapeDtypeStruct((N, D2), jnp.float32),
            jax.ShapeDtypeStruct((N, D2), jnp.float32),
        ),
    )(nf, wpq, bpq)


GRP = 200
GCH = 40
NKG = GRP // GCH
NG = EPW // GRP
NLG = GRP // 16


def _gather_body(p_hbm, q_hbm, cx_hbm, cy_hbm, cz_hbm, src_hbm, dst_hbm,
                 o1_hbm, o2_hbm, o3_hbm,
                 srcv, dstv, cxv, cyv, czv, bp, bq, bc,
                 sg0, sg1, sg2, sg3, sg4, sw):
    cid = lax.axis_index("c")
    sid = lax.axis_index("s")
    wid = sid * NC + cid
    base = wid * EPW
    sgs = [sg0, sg1, sg2, sg3, sg4]

    pltpu.sync_copy(src_hbm.at[pl.ds(base, EPW)], srcv.at[pl.ds(0, EPW)])
    pltpu.sync_copy(dst_hbm.at[pl.ds(base, EPW)], dstv.at[pl.ds(0, EPW)])
    pltpu.sync_copy(cx_hbm, cxv)
    pltpu.sync_copy(cy_hbm, cyv)
    pltpu.sync_copy(cz_hbm, czv)
    zero16 = jnp.zeros((16,), jnp.float32)
    for r in range(GRP + 16):
        bc[r, pl.ds(0, 16)] = zero16

    def group(g, carry):
        goff = g * GRP
        gds = []
        for k in range(NKG):
            idx_s = srcv.at[pl.ds(goff + k * GCH, GCH)]
            idx_d = dstv.at[pl.ds(goff + k * GCH, GCH)]
            gds.append((
                pltpu.async_copy(p_hbm.at[idx_s], bp.at[pl.ds(k * GCH, GCH)], sgs[k]),
                pltpu.async_copy(q_hbm.at[idx_d], bq.at[pl.ds(k * GCH, GCH)], sgs[k]),
            ))
        nmax = jnp.full((16,), N - 1, jnp.int32)
        zero = jnp.zeros((16,), jnp.int32)
        for j in range(NLG + 1):
            sv = srcv[pl.ds(goff + j * 16, 16)]
            dv = dstv[pl.ds(goff + j * 16, 16)]
            sv = jnp.minimum(jnp.maximum(sv, zero), nmax)
            dv = jnp.minimum(jnp.maximum(dv, zero), nmax)
            dx = plsc.load_gather(cxv, [sv]) - plsc.load_gather(cxv, [dv])
            dy = plsc.load_gather(cyv, [sv]) - plsc.load_gather(cyv, [dv])
            dz = plsc.load_gather(czv, [sv]) - plsc.load_gather(czv, [dv])
            radial = dx * dx + dy * dy + dz * dz
            rows = lax.iota(jnp.int32, 16) + (j * 16)
            plsc.store_scatter(bc, [rows, jnp.full((16,), 0, jnp.int32)], dx)
            plsc.store_scatter(bc, [rows, jnp.full((16,), 1, jnp.int32)], dy)
            plsc.store_scatter(bc, [rows, jnp.full((16,), 2, jnp.int32)], dz)
            plsc.store_scatter(bc, [rows, jnp.full((16,), 3, jnp.int32)], radial)
        wds = [pltpu.async_copy(
            bc.at[pl.ds(0, GRP)], o3_hbm.at[pl.ds(base + goff, GRP)], sw)]
        for k in range(NKG):
            dp, dq = gds[k]
            dp.wait()
            dq.wait()
            row0 = base + goff + k * GCH
            wds.append(pltpu.async_copy(
                bp.at[pl.ds(k * GCH, GCH)], o1_hbm.at[pl.ds(row0, GCH)], sw))
            wds.append(pltpu.async_copy(
                bq.at[pl.ds(k * GCH, GCH)], o2_hbm.at[pl.ds(row0, GCH)], sw))
        for w in wds:
            w.wait()
        return carry

    lax.fori_loop(0, NG, group, 0)


_gather_call = functools.partial(
    pl.kernel,
    out_type=(
        jax.ShapeDtypeStruct((E, D2), jnp.float32),
        jax.ShapeDtypeStruct((E, D2), jnp.float32),
        jax.ShapeDtypeStruct((E, 16), jnp.float32),
    ),
    mesh=plsc.VectorSubcoreMesh(
        core_axis_name="c", subcore_axis_name="s", num_cores=NC, num_subcores=NS),
    scratch_types=[
        pltpu.VMEM((EPW + 16, ), jnp.int32),
        pltpu.VMEM((EPW + 16, ), jnp.int32),
        pltpu.VMEM((N,), jnp.float32),
        pltpu.VMEM((N,), jnp.float32),
        pltpu.VMEM((N,), jnp.float32),
        pltpu.VMEM((GRP, D2), jnp.float32),
        pltpu.VMEM((GRP, D2), jnp.float32),
        pltpu.VMEM((GRP + 16, 16), jnp.float32),
        pltpu.SemaphoreType.DMA,
        pltpu.SemaphoreType.DMA,
        pltpu.SemaphoreType.DMA,
        pltpu.SemaphoreType.DMA,
        pltpu.SemaphoreType.DMA,
        pltpu.SemaphoreType.DMA,
    ],
    compiler_params=pltpu.CompilerParams(
        needs_layout_passes=False, use_tc_tiling_on_sc=False),
)(_gather_body)


def _edge_body(o1, o2, o3, ef, wcat, we2, be2, wc1, bc1, wc2, m_out):
    c3 = o3[...]
    radial = c3[:, 3:4]
    mm_in = jnp.concatenate([c3, ef[...]], axis=1)
    pre = _unpack_cols(o1[...]) + _unpack_cols(o2[...]) \
        + jnp.dot(mm_in, wcat[...], preferred_element_type=jnp.float32)
    m1 = _silu(pre)
    mh = _silu(jnp.dot(m1, we2[...], preferred_element_type=jnp.float32) + be2[...])
    c1 = _silu(jnp.dot(mh, wc1[...], preferred_element_type=jnp.float32) + bc1[...])
    coef = jnp.sum(c1 * wc2[...], axis=1, keepdims=True)
    s = coef / (jnp.sqrt(radial) + 1e-30)
    blk = mh.shape[0]
    m_out[:, :D] = mh
    m_out[:, D:] = jnp.concatenate(
        [c3[:, 0:3] * s, jnp.ones((blk, AW - D - 3), jnp.float32)], axis=1)


def _edge_call(o1, o2, o3, ef, wcat, we2, be2, wc1, bc1, wc2):
    blk = 4000
    grid = (E // blk,)
    full = lambda shape: pl.BlockSpec(shape, lambda i: (0, 0))
    return pl.pallas_call(
        _edge_body,
        grid=grid,
        in_specs=[
            pl.BlockSpec((blk, D2), lambda i: (i, 0)),
            pl.BlockSpec((blk, D2), lambda i: (i, 0)),
            pl.BlockSpec((blk, 16), lambda i: (i, 0)),
            pl.BlockSpec((blk, EF), lambda i: (i, 0)),
            full((2 * EF, H)),
            full((H, H)),
            full((1, H)),
            full((H, H)),
            full((1, H)),
            full((1, H)),
        ],
        out_specs=pl.BlockSpec((blk, AW), lambda i: (i, 0)),
        out_shape=jax.ShapeDtypeStruct((E, AW), jnp.float32),
        compiler_params=pltpu.CompilerParams(
            dimension_semantics=("arbitrary",)),
    )(o1, o2, o3, ef, wcat, we2, be2, wc1, bc1, wc2)


SCH = 40
SNCH = EPW // SCH
NKS = 5
NGS = SNCH // NKS


def _scatter_body(m_hbm, dst_hbm, zeros_hbm, out_hbm, dstv,
                  bm0, bm1, bm2, bm3, bm4, acc,
                  sr0, sr1, sr2, sr3, sr4, sw):
    cid = lax.axis_index("c")
    sid = lax.axis_index("s")
    wid = sid * NC + cid
    base = wid * EPW
    bms = [bm0, bm1, bm2, bm3, bm4]
    srs = [sr0, sr1, sr2, sr3, sr4]

    pltpu.sync_copy(zeros_hbm.at[pl.ds(sid * RPT, RPT)],
                    acc.at[pl.ds(sid * RPT, RPT)])
    plsc.subcore_barrier()

    def group(g, carry):
        rds = []
        for k in range(NKS):
            row0 = base + (g * NKS + k) * SCH
            rds.append(pltpu.async_copy(
                m_hbm.at[pl.ds(row0, SCH)], bms[k], srs[k]))
        pltpu.sync_copy(dst_hbm.at[wid, pl.ds(g * NKS, NKS)], dstv)
        ads = []
        for k in range(NKS):
            rds[k].wait()
            ads.append(pltpu.async_copy(
                bms[k], acc.at[dstv.at[k]], sw, add=True))
        for a in ads:
            a.wait()
        return carry

    lax.fori_loop(0, NGS, group, 0)
    plsc.subcore_barrier()
    pltpu.sync_copy(acc.at[pl.ds(sid * RPT, RPT)],
                    out_hbm.at[cid, pl.ds(sid * RPT, RPT)])


_scatter_call = functools.partial(
    pl.kernel,
    out_type=jax.ShapeDtypeStruct((NC, NP, AW), jnp.float32),
    mesh=plsc.VectorSubcoreMesh(
        core_axis_name="c", subcore_axis_name="s", num_cores=NC, num_subcores=NS),
    scratch_types=[
        pltpu.VMEM((NKS, SCH), jnp.int32),
        pltpu.VMEM((SCH, AW), jnp.float32),
        pltpu.VMEM((SCH, AW), jnp.float32),
        pltpu.VMEM((SCH, AW), jnp.float32),
        pltpu.VMEM((SCH, AW), jnp.float32),
        pltpu.VMEM((SCH, AW), jnp.float32),
        pltpu.VMEM_SHARED((NP, AW), jnp.float32),
        pltpu.SemaphoreType.DMA,
        pltpu.SemaphoreType.DMA,
        pltpu.SemaphoreType.DMA,
        pltpu.SemaphoreType.DMA,
        pltpu.SemaphoreType.DMA,
        pltpu.SemaphoreType.DMA,
    ],
    compiler_params=pltpu.CompilerParams(
        needs_layout_passes=False, use_tc_tiling_on_sc=False),
)(_scatter_body)


def _node_body(nf, cf, part, wn1, bn1, wn2, bn2, gam, bet, h_out, x_out):
    acc = part[0, :N] + part[1, :N]
    h_neigh = acc[:, :D]
    xsum = acc[:, D:D + 3]
    deg = acc[:, D + 3:D + 4]
    hin = jnp.concatenate([nf[...], h_neigh], axis=1)
    h1 = _silu(jnp.dot(hin, wn1[...], preferred_element_type=jnp.float32) + bn1[...])
    h2 = jnp.dot(h1, wn2[...], preferred_element_type=jnp.float32) + bn2[...]
    mean = jnp.mean(h2, axis=0, keepdims=True)
    var = jnp.mean(jnp.square(h2 - mean), axis=0, keepdims=True)
    h_out[...] = (h2 - mean) / jnp.sqrt(var + 1e-5) * gam[...] + bet[...]
    x_out[...] = cf[...] + xsum / jnp.maximum(deg, 1.0)


def _node_call(nf, cf, part, wn1, bn1, wn2, bn2, gam, bet):
    return pl.pallas_call(
        _node_body,
        out_shape=(
            jax.ShapeDtypeStruct((N, O_), jnp.float32),
            jax.ShapeDtypeStruct((N, 3), jnp.float32),
        ),
    )(nf, cf, part, wn1, bn1, wn2, bn2, gam, bet)


@jax.jit
def kernel(node_feat, coord_feat, edge_index, edge_feat, W_e1, b_e1, W_e2,
           b_e2, W_n1, b_n1, W_n2, b_n2, W_c1, b_c1, W_c2, bn_gamma, bn_beta):
    src = edge_index[0]
    dst = edge_index[1]
    dst3 = dst.reshape(NW, SNCH, SCH)
    cx = coord_feat[:, 0]
    cy = coord_feat[:, 1]
    cz = coord_feat[:, 2]

    wpq = jnp.concatenate([W_e1[:D], W_e1[D:2 * D]], axis=1)
    bpq = jnp.concatenate([jnp.zeros((H,), jnp.float32), b_e1]).reshape(1, 2 * H)
    wcat = jnp.concatenate([
        jnp.zeros((3, H), jnp.float32),
        W_e1[2 * D:2 * D + 1],
        jnp.zeros((12, H), jnp.float32),
        W_e1[2 * D + 1:],
    ], axis=0)

    p, q = _pq_call(node_feat, wpq, bpq)
    o1, o2, o3 = _gather_call(p, q, cx, cy, cz, src, dst)
    m = _edge_call(o1, o2, o3, edge_feat, wcat, W_e2, b_e2.reshape(1, H),
                   W_c1, b_c1.reshape(1, H), W_c2.reshape(1, H))
    part = _scatter_call(m, dst3, jnp.zeros((NP, AW), jnp.float32))
    h, x = _node_call(node_feat, coord_feat, part, W_n1, b_n1.reshape(1, H),
                      W_n2, b_n2.reshape(1, O_), bn_gamma.reshape(1, O_),
                      bn_beta.reshape(1, O_))
    return (h, x)


# speedup vs baseline: 1.3174x; 1.3174x over previous
import functools

import jax
import jax.numpy as jnp
from jax import lax
from jax.experimental import pallas as pl
from jax.experimental.pallas import tpu as pltpu
from jax.experimental.pallas import tpu_sc as plsc

N = 10000
E = 320000
D = 128
H = 128
O_ = 128
EF = 16

NC = 2
NS = 16
NW = NC * NS
EPW = E // NW
CH = 80
NCH = EPW // CH
NP = 10240
RPT = NP // NS
AW = 144


def _silu(x):
    return x / (1.0 + jnp.exp(-x))


D2 = D // 2


def _pack_cols(x_lo, x_hi):
    lo = jax.lax.bitcast_convert_type(
        x_lo.astype(jnp.bfloat16), jnp.uint16).astype(jnp.uint32)
    hi = jax.lax.bitcast_convert_type(
        x_hi.astype(jnp.bfloat16), jnp.uint16).astype(jnp.uint32)
    return jax.lax.bitcast_convert_type((hi << 16) | lo, jnp.float32)


def _unpack_cols(xp):
    w = jax.lax.bitcast_convert_type(xp, jnp.uint32)
    lo = jax.lax.bitcast_convert_type(w << 16, jnp.float32)
    hi = jax.lax.bitcast_convert_type(w & jnp.uint32(0xFFFF0000), jnp.float32)
    return jnp.concatenate([lo, hi], axis=1)


def _pq_body(nf, wpq, bpq, p_out, q_out):
    pq = jnp.dot(nf[...], wpq[...], preferred_element_type=jnp.float32)
    pq = pq + bpq[...]
    p_out[...] = _pack_cols(pq[:, 0:D2], pq[:, D2:D])
    q_out[...] = _pack_cols(pq[:, D:D + D2], pq[:, D + D2:])


def _pq_call(nf, wpq, bpq):
    return pl.pallas_call(
        _pq_body,
        out_shape=(
            jax.ShapeDtypeStruct((N, D2), jnp.float32),
            jax.ShapeDtypeStruct((N, D2), jnp.float32),
        ),
    )(nf, wpq, bpq)


GRP = 200
GCH = 40
NKG = GRP // GCH
NG = EPW // GRP
NLG = GRP // 16


def _gather_body(p_hbm, q_hbm, cx_hbm, cy_hbm, cz_hbm, src_hbm, dst_hbm,
                 o12_hbm, dx_hbm, dy_hbm, dz_hbm, rad_hbm,
                 srcv, dstv, cxv, cyv, czv, bp, bq, bct,
                 sg0, sg1, sg2, sg3, sg4, sw):
    cid = lax.axis_index("c")
    sid = lax.axis_index("s")
    wid = sid * NC + cid
    base = wid * EPW
    sgs = [sg0, sg1, sg2, sg3, sg4]

    pltpu.sync_copy(src_hbm.at[pl.ds(base, EPW)], srcv.at[pl.ds(0, EPW)])
    pltpu.sync_copy(dst_hbm.at[pl.ds(base, EPW)], dstv.at[pl.ds(0, EPW)])
    pltpu.sync_copy(cx_hbm, cxv)
    pltpu.sync_copy(cy_hbm, cyv)
    pltpu.sync_copy(cz_hbm, czv)

    def group(g, carry):
        goff = g * GRP
        gds = []
        for k in range(NKG):
            idx_s = srcv.at[pl.ds(goff + k * GCH, GCH)]
            idx_d = dstv.at[pl.ds(goff + k * GCH, GCH)]
            gds.append((
                pltpu.async_copy(
                    p_hbm.at[idx_s], bp.at[pl.ds(k * GCH, GCH)], sgs[k]),
                pltpu.async_copy(
                    q_hbm.at[idx_d], bq.at[pl.ds(k * GCH, GCH)], sgs[k]),
            ))
        nmax = jnp.full((16,), N - 1, jnp.int32)
        zero = jnp.zeros((16,), jnp.int32)
        for j in range(NLG + 1):
            sv = srcv[pl.ds(goff + j * 16, 16)]
            dv = dstv[pl.ds(goff + j * 16, 16)]
            sv = jnp.minimum(jnp.maximum(sv, zero), nmax)
            dv = jnp.minimum(jnp.maximum(dv, zero), nmax)
            dx = plsc.load_gather(cxv, [sv]) - plsc.load_gather(cxv, [dv])
            dy = plsc.load_gather(cyv, [sv]) - plsc.load_gather(cyv, [dv])
            dz = plsc.load_gather(czv, [sv]) - plsc.load_gather(czv, [dv])
            radial = dx * dx + dy * dy + dz * dz
            cols = lax.iota(jnp.int32, 16) + (j * 16)
            plsc.store_scatter(bct, [jnp.full((16,), 0, jnp.int32), cols], dx)
            plsc.store_scatter(bct, [jnp.full((16,), 1, jnp.int32), cols], dy)
            plsc.store_scatter(bct, [jnp.full((16,), 2, jnp.int32), cols], dz)
            plsc.store_scatter(bct, [jnp.full((16,), 3, jnp.int32), cols], radial)
        for k in range(NKG):
            dp, dq = gds[k]
            dp.wait()
            dq.wait()
        orows = o12_hbm.at[pl.ds(base + goff, GRP)]
        erows = pl.ds(base + goff, GRP)
        wds = [
            pltpu.async_copy(bp, orows.at[:, pl.ds(0, D2)], sw),
            pltpu.async_copy(bq, orows.at[:, pl.ds(D2, D2)], sw),
            pltpu.async_copy(bct.at[0, pl.ds(0, GRP)], dx_hbm.at[erows], sw),
            pltpu.async_copy(bct.at[1, pl.ds(0, GRP)], dy_hbm.at[erows], sw),
            pltpu.async_copy(bct.at[2, pl.ds(0, GRP)], dz_hbm.at[erows], sw),
            pltpu.async_copy(bct.at[3, pl.ds(0, GRP)], rad_hbm.at[erows], sw),
        ]
        for w in wds:
            w.wait()
        return carry

    lax.fori_loop(0, NG, group, 0)


_gather_call = functools.partial(
    pl.kernel,
    out_type=(
        jax.ShapeDtypeStruct((E, D), jnp.float32),
        jax.ShapeDtypeStruct((E,), jnp.float32),
        jax.ShapeDtypeStruct((E,), jnp.float32),
        jax.ShapeDtypeStruct((E,), jnp.float32),
        jax.ShapeDtypeStruct((E,), jnp.float32),
    ),
    mesh=plsc.VectorSubcoreMesh(
        core_axis_name="c", subcore_axis_name="s", num_cores=NC, num_subcores=NS),
    scratch_types=[
        pltpu.VMEM((EPW + 16, ), jnp.int32),
        pltpu.VMEM((EPW + 16, ), jnp.int32),
        pltpu.VMEM((N,), jnp.float32),
        pltpu.VMEM((N,), jnp.float32),
        pltpu.VMEM((N,), jnp.float32),
        pltpu.VMEM((GRP, D2), jnp.float32),
        pltpu.VMEM((GRP, D2), jnp.float32),
        pltpu.VMEM((4, GRP + 16), jnp.float32),
        pltpu.SemaphoreType.DMA,
        pltpu.SemaphoreType.DMA,
        pltpu.SemaphoreType.DMA,
        pltpu.SemaphoreType.DMA,
        pltpu.SemaphoreType.DMA,
        pltpu.SemaphoreType.DMA,
    ],
    compiler_params=pltpu.CompilerParams(
        needs_layout_passes=False, use_tc_tiling_on_sc=False),
)(_gather_body)


def _edge_body(o12, rad, ef, wr, wef, we2, be2, wc1, bc1, wc2, m_out, s_out):
    ob = o12[...]
    rad1 = rad[...]
    blk = ob.shape[0]
    pre = _unpack_cols(ob[:, :D2]) + _unpack_cols(ob[:, D2:]) \
        + lax.dot_general(rad1.reshape(1, blk), wr[...],
                          (((0,), (0,)), ((), ())),
                          preferred_element_type=jnp.float32) \
        + jnp.dot(ef[...], wef[...], preferred_element_type=jnp.float32)
    m1 = _silu(pre)
    mh = _silu(jnp.dot(m1, we2[...], preferred_element_type=jnp.float32) + be2[...])
    c1 = _silu(jnp.dot(mh, wc1[...], preferred_element_type=jnp.float32) + bc1[...])
    coef = jnp.dot(c1, wc2[...], preferred_element_type=jnp.float32)
    m_out[...] = mh
    s_out[...] = coef / (jnp.sqrt(rad1) + 1e-30)


def _edge_call(o12, rad, ef, wr, wef, we2, be2, wc1, bc1, wc2):
    blk = 512
    grid = (E // blk,)
    full = lambda shape: pl.BlockSpec(shape, lambda i: (0,) * len(shape))
    return pl.pallas_call(
        _edge_body,
        grid=grid,
        in_specs=[
            pl.BlockSpec((blk, D), lambda i: (i, 0)),
            pl.BlockSpec((blk,), lambda i: (i,)),
            pl.BlockSpec((blk, EF), lambda i: (i, 0)),
            full((1, H)),
            full((EF, H)),
            full((H, H)),
            full((1, H)),
            full((H, H)),
            full((1, H)),
            full((H,)),
        ],
        out_specs=(
            pl.BlockSpec((blk, D), lambda i: (i, 0)),
            pl.BlockSpec((blk,), lambda i: (i,)),
        ),
        out_shape=(
            jax.ShapeDtypeStruct((E, D), jnp.float32),
            jax.ShapeDtypeStruct((E,), jnp.float32),
        ),
        compiler_params=pltpu.CompilerParams(
            dimension_semantics=("arbitrary",)),
    )(o12, rad, ef, wr, wef, we2, be2, wc1, bc1, wc2)


SCH = 40
SNCH = EPW // SCH
NKS = 5
NGS = SNCH // NKS


def _scath_body(m_hbm, dst_hbm, zeros_hbm, out_hbm, dstv,
                bm0, bm1, bm2, bm3, bm4, acc,
                sr0, sr1, sr2, sr3, sr4, sw):
    cid = lax.axis_index("c")
    sid = lax.axis_index("s")
    wid = sid * NC + cid
    base = wid * EPW
    bms = [bm0, bm1, bm2, bm3, bm4]
    srs = [sr0, sr1, sr2, sr3, sr4]

    pltpu.sync_copy(zeros_hbm.at[pl.ds(sid * RPT, RPT)],
                    acc.at[pl.ds(sid * RPT, RPT)])
    plsc.subcore_barrier()

    def group(g, carry):
        rds = []
        for k in range(NKS):
            row0 = base + (g * NKS + k) * SCH
            rds.append(pltpu.async_copy(
                m_hbm.at[pl.ds(row0, SCH)], bms[k], srs[k]))
        pltpu.sync_copy(dst_hbm.at[wid, g], dstv)
        ads = []
        for k in range(NKS):
            rds[k].wait()
            ads.append(pltpu.async_copy(
                bms[k], acc.at[dstv.at[k]], sw, add=True))
        for a in ads:
            a.wait()
        return carry

    lax.fori_loop(0, NGS, group, 0)
    plsc.subcore_barrier()
    pltpu.sync_copy(acc.at[pl.ds(sid * RPT, RPT)],
                    out_hbm.at[cid, pl.ds(sid * RPT, RPT)])


_scath_call = functools.partial(
    pl.kernel,
    out_type=jax.ShapeDtypeStruct((NC, NP, D), jnp.float32),
    mesh=plsc.VectorSubcoreMesh(
        core_axis_name="c", subcore_axis_name="s", num_cores=NC, num_subcores=NS),
    scratch_types=[
        pltpu.VMEM((NKS, SCH), jnp.int32),
        pltpu.VMEM((SCH, D), jnp.float32),
        pltpu.VMEM((SCH, D), jnp.float32),
        pltpu.VMEM((SCH, D), jnp.float32),
        pltpu.VMEM((SCH, D), jnp.float32),
        pltpu.VMEM((SCH, D), jnp.float32),
        pltpu.VMEM_SHARED((NP, D), jnp.float32),
        pltpu.SemaphoreType.DMA,
        pltpu.SemaphoreType.DMA,
        pltpu.SemaphoreType.DMA,
        pltpu.SemaphoreType.DMA,
        pltpu.SemaphoreType.DMA,
        pltpu.SemaphoreType.DMA,
    ],
    compiler_params=pltpu.CompilerParams(needs_layout_passes=False),
)(_scath_body)


SCH2 = 80
NCH2 = EPW // SCH2
NKX = 5
NGX = NCH2 // NKX
GX = SCH2 * NKX


def _scatx_body(s_hbm, dx_hbm, dy_hbm, dz_hbm, dst_hbm, zeros_hbm, out_hbm,
                dstv, bs, bx, by, bz, baux, acc, sl, sw):
    cid = lax.axis_index("c")
    sid = lax.axis_index("s")
    wid = sid * NC + cid
    base = wid * EPW

    pltpu.sync_copy(zeros_hbm.at[pl.ds(sid * RPT, RPT)],
                    acc.at[pl.ds(sid * RPT, RPT)])
    pltpu.sync_copy(dst_hbm.at[wid], dstv)
    plsc.subcore_barrier()
    unit = jnp.where(lax.iota(jnp.int32, 16) == 3,
                     jnp.full((16,), 1.0, jnp.float32),
                     jnp.zeros((16,), jnp.float32))

    def prow(r, carry):
        baux[r, pl.ds(0, 16)] = unit
        return carry

    lax.fori_loop(0, GX, prow, 0)

    def group(g, carry):
        goff = base + g * GX
        lds = [
            pltpu.async_copy(s_hbm.at[pl.ds(goff, GX)], bs, sl),
            pltpu.async_copy(dx_hbm.at[pl.ds(goff, GX)], bx, sl),
            pltpu.async_copy(dy_hbm.at[pl.ds(goff, GX)], by, sl),
            pltpu.async_copy(dz_hbm.at[pl.ds(goff, GX)], bz, sl),
        ]
        for l in lds:
            l.wait()
        for j in range(GX // 16):
            sv = bs[pl.ds(j * 16, 16)]
            rows = lax.iota(jnp.int32, 16) + (j * 16)
            plsc.store_scatter(
                baux, [rows, jnp.full((16,), 0, jnp.int32)],
                sv * bx[pl.ds(j * 16, 16)])
            plsc.store_scatter(
                baux, [rows, jnp.full((16,), 1, jnp.int32)],
                sv * by[pl.ds(j * 16, 16)])
            plsc.store_scatter(
                baux, [rows, jnp.full((16,), 2, jnp.int32)],
                sv * bz[pl.ds(j * 16, 16)])
        ads = []
        for k in range(NKX):
            ads.append(pltpu.async_copy(
                baux.at[pl.ds(k * SCH2, SCH2)],
                acc.at[dstv.at[g * NKX + k]], sw, add=True))
        for a in ads:
            a.wait()
        return carry

    lax.fori_loop(0, NGX, group, 0)
    plsc.subcore_barrier()
    pltpu.sync_copy(acc.at[pl.ds(sid * RPT, RPT)],
                    out_hbm.at[cid, pl.ds(sid * RPT, RPT)])


_scatx_call = functools.partial(
    pl.kernel,
    out_type=jax.ShapeDtypeStruct((NC, NP, 16), jnp.float32),
    mesh=plsc.VectorSubcoreMesh(
        core_axis_name="c", subcore_axis_name="s", num_cores=NC, num_subcores=NS),
    scratch_types=[
        pltpu.VMEM((NCH2, SCH2), jnp.int32),
        pltpu.VMEM((GX,), jnp.float32),
        pltpu.VMEM((GX,), jnp.float32),
        pltpu.VMEM((GX,), jnp.float32),
        pltpu.VMEM((GX,), jnp.float32),
        pltpu.VMEM((GX, 16), jnp.float32),
        pltpu.VMEM_SHARED((NP, 16), jnp.float32),
        pltpu.SemaphoreType.DMA,
        pltpu.SemaphoreType.DMA,
    ],
    compiler_params=pltpu.CompilerParams(
        needs_layout_passes=False, use_tc_tiling_on_sc=False),
)(_scatx_body)


def _node_body(nf, cf, part, paux, wn1, bn1, wn2, bn2, gam, bet, h_out, x_out):
    h_neigh = part[0, :N] + part[1, :N]
    aux = paux[0, :N] + paux[1, :N]
    xsum = aux[:, 0:3]
    deg = aux[:, 3:4]
    hin = jnp.concatenate([nf[...], h_neigh], axis=1)
    h1 = _silu(jnp.dot(hin, wn1[...], preferred_element_type=jnp.float32) + bn1[...])
    h2 = jnp.dot(h1, wn2[...], preferred_element_type=jnp.float32) + bn2[...]
    mean = jnp.mean(h2, axis=0, keepdims=True)
    var = jnp.mean(jnp.square(h2 - mean), axis=0, keepdims=True)
    h_out[...] = (h2 - mean) / jnp.sqrt(var + 1e-5) * gam[...] + bet[...]
    x_out[...] = cf[...] + xsum / jnp.maximum(deg, 1.0)


def _node_call(nf, cf, part, paux, wn1, bn1, wn2, bn2, gam, bet):
    return pl.pallas_call(
        _node_body,
        out_shape=(
            jax.ShapeDtypeStruct((N, O_), jnp.float32),
            jax.ShapeDtypeStruct((N, 3), jnp.float32),
        ),
    )(nf, cf, part, paux, wn1, bn1, wn2, bn2, gam, bet)


@jax.jit
def kernel(node_feat, coord_feat, edge_index, edge_feat, W_e1, b_e1, W_e2,
           b_e2, W_n1, b_n1, W_n2, b_n2, W_c1, b_c1, W_c2, bn_gamma, bn_beta):
    src = edge_index[0]
    dst = edge_index[1]
    dst3a = dst.reshape(NW, NGS, NKS, SCH)
    dst3b = dst.reshape(NW, NCH2, SCH2)
    cx = coord_feat[:, 0]
    cy = coord_feat[:, 1]
    cz = coord_feat[:, 2]

    wpq = jnp.concatenate([W_e1[:D], W_e1[D:2 * D]], axis=1)
    bpq = jnp.concatenate([jnp.zeros((H,), jnp.float32), b_e1]).reshape(1, 2 * H)
    wr = W_e1[2 * D:2 * D + 1]
    wef = W_e1[2 * D + 1:]

    p, q = _pq_call(node_feat, wpq, bpq)
    o12, dxo, dyo, dzo, rado = _gather_call(p, q, cx, cy, cz, src, dst)
    m, s = _edge_call(o12, rado, edge_feat, wr, wef, W_e2, b_e2.reshape(1, H),
                      W_c1, b_c1.reshape(1, H), W_c2.reshape(H))
    part = _scath_call(m, dst3a, jnp.zeros((NP, D), jnp.float32))
    paux = _scatx_call(s, dxo, dyo, dzo, dst3b,
                       jnp.zeros((NP, 16), jnp.float32))
    h, x = _node_call(node_feat, coord_feat, part, paux, W_n1,
                      b_n1.reshape(1, H), W_n2, b_n2.reshape(1, O_),
                      bn_gamma.reshape(1, O_), bn_beta.reshape(1, O_))
    return (h, x)


# speedup vs baseline: 1.7713x; 1.3446x over previous
import functools

import jax
import jax.numpy as jnp
from jax import lax
from jax.experimental import pallas as pl
from jax.experimental.pallas import tpu as pltpu
from jax.experimental.pallas import tpu_sc as plsc

N = 10000
E = 320000
EP = 327680
D = 128
H = 128
O_ = 128
EF = 16

NC = 2
NS = 16
NW = NC * NS
EPW = E // NW
CH = 80
NCH = EPW // CH
NP = 10240
RPT = NP // NS
AW = 144


def _silu(x):
    return x / (1.0 + jnp.exp(-x))


D2 = D // 2


def _pack_cols(x_lo, x_hi):
    lo = jax.lax.bitcast_convert_type(
        x_lo.astype(jnp.bfloat16), jnp.uint16).astype(jnp.uint32)
    hi = jax.lax.bitcast_convert_type(
        x_hi.astype(jnp.bfloat16), jnp.uint16).astype(jnp.uint32)
    return jax.lax.bitcast_convert_type((hi << 16) | lo, jnp.float32)


def _unpack_cols(xp):
    w = jax.lax.bitcast_convert_type(xp, jnp.uint32)
    lo = jax.lax.bitcast_convert_type(w << 16, jnp.float32)
    hi = jax.lax.bitcast_convert_type(w & jnp.uint32(0xFFFF0000), jnp.float32)
    return jnp.concatenate([lo, hi], axis=1)


def _pq_body(nf, wpq, bpq, p_out, q_out):
    pq = jnp.dot(nf[...], wpq[...], preferred_element_type=jnp.float32)
    pq = pq + bpq[...]
    p_out[...] = _pack_cols(pq[:, 0:D2], pq[:, D2:D])
    q_out[...] = _pack_cols(pq[:, D:D + D2], pq[:, D + D2:])


def _pq_call(nf, wpq, bpq):
    return pl.pallas_call(
        _pq_body,
        out_shape=(
            jax.ShapeDtypeStruct((N, D2), jnp.float32),
            jax.ShapeDtypeStruct((N, D2), jnp.float32),
        ),
    )(nf, wpq, bpq)


GRP = 200
GCH = 40
NKG = GRP // GCH
NG = EPW // GRP
NLG = GRP // 16


def _gather_body(p_hbm, q_hbm, cx_hbm, cy_hbm, cz_hbm, src_hbm, dst_hbm,
                 o12_hbm, dx_hbm, dy_hbm, dz_hbm, rad_hbm,
                 srcv, dstv, cxv, cyv, czv, bp, bq, bct,
                 sg0, sg1, sg2, sg3, sg4, sw):
    cid = lax.axis_index("c")
    sid = lax.axis_index("s")
    wid = sid * NC + cid
    base = wid * EPW
    sgs = [sg0, sg1, sg2, sg3, sg4]

    pltpu.sync_copy(src_hbm.at[pl.ds(base, EPW)], srcv.at[pl.ds(0, EPW)])
    pltpu.sync_copy(dst_hbm.at[pl.ds(base, EPW)], dstv.at[pl.ds(0, EPW)])
    pltpu.sync_copy(cx_hbm, cxv)
    pltpu.sync_copy(cy_hbm, cyv)
    pltpu.sync_copy(cz_hbm, czv)

    def group(g, carry):
        goff = g * GRP
        gds = []
        for k in range(NKG):
            idx_s = srcv.at[pl.ds(goff + k * GCH, GCH)]
            idx_d = dstv.at[pl.ds(goff + k * GCH, GCH)]
            gds.append((
                pltpu.async_copy(
                    p_hbm.at[idx_s], bp.at[pl.ds(k * GCH, GCH)], sgs[k]),
                pltpu.async_copy(
                    q_hbm.at[idx_d], bq.at[pl.ds(k * GCH, GCH)], sgs[k]),
            ))
        nmax = jnp.full((16,), N - 1, jnp.int32)
        zero = jnp.zeros((16,), jnp.int32)
        for j in range(NLG + 1):
            sv = srcv[pl.ds(goff + j * 16, 16)]
            dv = dstv[pl.ds(goff + j * 16, 16)]
            sv = jnp.minimum(jnp.maximum(sv, zero), nmax)
            dv = jnp.minimum(jnp.maximum(dv, zero), nmax)
            dx = plsc.load_gather(cxv, [sv]) - plsc.load_gather(cxv, [dv])
            dy = plsc.load_gather(cyv, [sv]) - plsc.load_gather(cyv, [dv])
            dz = plsc.load_gather(czv, [sv]) - plsc.load_gather(czv, [dv])
            radial = dx * dx + dy * dy + dz * dz
            cols = lax.iota(jnp.int32, 16) + (j * 16)
            plsc.store_scatter(bct, [jnp.full((16,), 0, jnp.int32), cols], dx)
            plsc.store_scatter(bct, [jnp.full((16,), 1, jnp.int32), cols], dy)
            plsc.store_scatter(bct, [jnp.full((16,), 2, jnp.int32), cols], dz)
            plsc.store_scatter(bct, [jnp.full((16,), 3, jnp.int32), cols], radial)
        for k in range(NKG):
            dp, dq = gds[k]
            dp.wait()
            dq.wait()
        orows = o12_hbm.at[pl.ds(base + goff, GRP)]
        erows = pl.ds(base + goff, GRP)
        wds = [
            pltpu.async_copy(bp, orows.at[:, pl.ds(0, D2)], sw),
            pltpu.async_copy(bq, orows.at[:, pl.ds(D2, D2)], sw),
            pltpu.async_copy(bct.at[0, pl.ds(0, GRP)], dx_hbm.at[erows], sw),
            pltpu.async_copy(bct.at[1, pl.ds(0, GRP)], dy_hbm.at[erows], sw),
            pltpu.async_copy(bct.at[2, pl.ds(0, GRP)], dz_hbm.at[erows], sw),
            pltpu.async_copy(bct.at[3, pl.ds(0, GRP)], rad_hbm.at[erows], sw),
        ]
        for w in wds:
            w.wait()
        return carry

    lax.fori_loop(0, NG, group, 0)


_gather_call = functools.partial(
    pl.kernel,
    out_type=(
        jax.ShapeDtypeStruct((EP, D), jnp.float32),
        jax.ShapeDtypeStruct((EP,), jnp.float32),
        jax.ShapeDtypeStruct((EP,), jnp.float32),
        jax.ShapeDtypeStruct((EP,), jnp.float32),
        jax.ShapeDtypeStruct((EP,), jnp.float32),
    ),
    mesh=plsc.VectorSubcoreMesh(
        core_axis_name="c", subcore_axis_name="s", num_cores=NC, num_subcores=NS),
    scratch_types=[
        pltpu.VMEM((EPW + 16, ), jnp.int32),
        pltpu.VMEM((EPW + 16, ), jnp.int32),
        pltpu.VMEM((N,), jnp.float32),
        pltpu.VMEM((N,), jnp.float32),
        pltpu.VMEM((N,), jnp.float32),
        pltpu.VMEM((GRP, D2), jnp.float32),
        pltpu.VMEM((GRP, D2), jnp.float32),
        pltpu.VMEM((4, GRP + 16), jnp.float32),
        pltpu.SemaphoreType.DMA,
        pltpu.SemaphoreType.DMA,
        pltpu.SemaphoreType.DMA,
        pltpu.SemaphoreType.DMA,
        pltpu.SemaphoreType.DMA,
        pltpu.SemaphoreType.DMA,
    ],
    compiler_params=pltpu.CompilerParams(
        needs_layout_passes=False, use_tc_tiling_on_sc=False),
)(_gather_body)


EBLK = 4096
ERW = EBLK // 128


def _edge_body(o12, rad, ef, wr, wef, we2, be2, wc1, bc1, wc2, m_out, s_out):
    ob = o12[...]
    rad_row = rad[...].reshape(1, EBLK)
    pre = _unpack_cols(ob[:, :D2]) + _unpack_cols(ob[:, D2:]) \
        + lax.dot_general(rad_row, wr[...],
                          (((0,), (0,)), ((), ())),
                          preferred_element_type=jnp.float32) \
        + jnp.dot(ef[...], wef[...], preferred_element_type=jnp.float32)
    m1 = _silu(pre)
    mh = _silu(jnp.dot(m1, we2[...], preferred_element_type=jnp.float32) + be2[...])
    c1 = _silu(jnp.dot(mh, wc1[...], preferred_element_type=jnp.float32) + bc1[...])
    coef = jnp.dot(c1, wc2[...], preferred_element_type=jnp.float32)
    m_out[...] = mh
    s_row = coef.reshape(1, EBLK) / (jnp.sqrt(rad_row) + 1e-30)
    s_out[...] = s_row.reshape(ERW, 128)


def _edge_call(o12, rad, ef, wr, wef, we2, be2, wc1, bc1, wc2):
    grid = (EP // EBLK,)
    full = lambda shape: pl.BlockSpec(shape, lambda i: (0,) * len(shape))
    return pl.pallas_call(
        _edge_body,
        grid=grid,
        in_specs=[
            pl.BlockSpec((EBLK, D), lambda i: (i, 0)),
            pl.BlockSpec((ERW, 128), lambda i: (i, 0)),
            pl.BlockSpec((EBLK, EF), lambda i: (i, 0)),
            full((1, H)),
            full((EF, H)),
            full((H, H)),
            full((1, H)),
            full((H, H)),
            full((1, H)),
            full((H,)),
        ],
        out_specs=(
            pl.BlockSpec((EBLK, D), lambda i: (i, 0)),
            pl.BlockSpec((ERW, 128), lambda i: (i, 0)),
        ),
        out_shape=(
            jax.ShapeDtypeStruct((EP, D), jnp.float32),
            jax.ShapeDtypeStruct((EP // 128, 128), jnp.float32),
        ),
        compiler_params=pltpu.CompilerParams(
            dimension_semantics=("arbitrary",)),
    )(o12, rad, ef, wr, wef, we2, be2, wc1, bc1, wc2)


SCH = 40
SNCH = EPW // SCH
NKS = 5
NGS = SNCH // NKS


def _scath_body(m_hbm, dst_hbm, zeros_hbm, out_hbm, dstv,
                bm0, bm1, bm2, bm3, bm4, acc,
                sr0, sr1, sr2, sr3, sr4, sw):
    cid = lax.axis_index("c")
    sid = lax.axis_index("s")
    wid = sid * NC + cid
    base = wid * EPW
    bms = [bm0, bm1, bm2, bm3, bm4]
    srs = [sr0, sr1, sr2, sr3, sr4]

    pltpu.sync_copy(zeros_hbm.at[pl.ds(sid * RPT, RPT)],
                    acc.at[pl.ds(sid * RPT, RPT)])
    plsc.subcore_barrier()

    def group(g, carry):
        rds = []
        for k in range(NKS):
            row0 = base + (g * NKS + k) * SCH
            rds.append(pltpu.async_copy(
                m_hbm.at[pl.ds(row0, SCH)], bms[k], srs[k]))
        pltpu.sync_copy(dst_hbm.at[wid, g], dstv)
        ads = []
        for k in range(NKS):
            rds[k].wait()
            ads.append(pltpu.async_copy(
                bms[k], acc.at[dstv.at[k]], sw, add=True))
        for a in ads:
            a.wait()
        return carry

    lax.fori_loop(0, NGS, group, 0)
    plsc.subcore_barrier()
    pltpu.sync_copy(acc.at[pl.ds(sid * RPT, RPT)],
                    out_hbm.at[cid, pl.ds(sid * RPT, RPT)])


_scath_call = functools.partial(
    pl.kernel,
    out_type=jax.ShapeDtypeStruct((NC, NP, D), jnp.float32),
    mesh=plsc.VectorSubcoreMesh(
        core_axis_name="c", subcore_axis_name="s", num_cores=NC, num_subcores=NS),
    scratch_types=[
        pltpu.VMEM((NKS, SCH), jnp.int32),
        pltpu.VMEM((SCH, D), jnp.float32),
        pltpu.VMEM((SCH, D), jnp.float32),
        pltpu.VMEM((SCH, D), jnp.float32),
        pltpu.VMEM((SCH, D), jnp.float32),
        pltpu.VMEM((SCH, D), jnp.float32),
        pltpu.VMEM_SHARED((NP, D), jnp.float32),
        pltpu.SemaphoreType.DMA,
        pltpu.SemaphoreType.DMA,
        pltpu.SemaphoreType.DMA,
        pltpu.SemaphoreType.DMA,
        pltpu.SemaphoreType.DMA,
        pltpu.SemaphoreType.DMA,
    ],
    compiler_params=pltpu.CompilerParams(needs_layout_passes=False),
)(_scath_body)


SCH2 = 80
NCH2 = EPW // SCH2
NKX = 5
NGX = NCH2 // NKX
GX = SCH2 * NKX


def _scatx_body(s_hbm, dx_hbm, dy_hbm, dz_hbm, dst_hbm, zeros_hbm, out_hbm,
                dstv, bs, bx, by, bz, baux, acc, sl, sw):
    cid = lax.axis_index("c")
    sid = lax.axis_index("s")
    wid = sid * NC + cid
    base = wid * EPW

    pltpu.sync_copy(zeros_hbm.at[pl.ds(sid * RPT, RPT)],
                    acc.at[pl.ds(sid * RPT, RPT)])
    pltpu.sync_copy(dst_hbm.at[wid], dstv)
    plsc.subcore_barrier()
    unit = jnp.where(lax.iota(jnp.int32, 16) == 3,
                     jnp.full((16,), 1.0, jnp.float32),
                     jnp.zeros((16,), jnp.float32))

    def prow(r, carry):
        baux[r, pl.ds(0, 16)] = unit
        return carry

    lax.fori_loop(0, GX, prow, 0)

    def group(g, carry):
        goff = base + g * GX
        lds = [
            pltpu.async_copy(s_hbm.at[pl.ds(goff, GX)], bs, sl),
            pltpu.async_copy(dx_hbm.at[pl.ds(goff, GX)], bx, sl),
            pltpu.async_copy(dy_hbm.at[pl.ds(goff, GX)], by, sl),
            pltpu.async_copy(dz_hbm.at[pl.ds(goff, GX)], bz, sl),
        ]
        for l in lds:
            l.wait()
        for j in range(GX // 16):
            sv = bs[pl.ds(j * 16, 16)]
            rows = lax.iota(jnp.int32, 16) + (j * 16)
            plsc.store_scatter(
                baux, [rows, jnp.full((16,), 0, jnp.int32)],
                sv * bx[pl.ds(j * 16, 16)])
            plsc.store_scatter(
                baux, [rows, jnp.full((16,), 1, jnp.int32)],
                sv * by[pl.ds(j * 16, 16)])
            plsc.store_scatter(
                baux, [rows, jnp.full((16,), 2, jnp.int32)],
                sv * bz[pl.ds(j * 16, 16)])
        ads = []
        for k in range(NKX):
            ads.append(pltpu.async_copy(
                baux.at[pl.ds(k * SCH2, SCH2)],
                acc.at[dstv.at[g * NKX + k]], sw, add=True))
        for a in ads:
            a.wait()
        return carry

    lax.fori_loop(0, NGX, group, 0)
    plsc.subcore_barrier()
    pltpu.sync_copy(acc.at[pl.ds(sid * RPT, RPT)],
                    out_hbm.at[cid, pl.ds(sid * RPT, RPT)])


_scatx_call = functools.partial(
    pl.kernel,
    out_type=jax.ShapeDtypeStruct((NC, NP, 16), jnp.float32),
    mesh=plsc.VectorSubcoreMesh(
        core_axis_name="c", subcore_axis_name="s", num_cores=NC, num_subcores=NS),
    scratch_types=[
        pltpu.VMEM((NCH2, SCH2), jnp.int32),
        pltpu.VMEM((GX,), jnp.float32),
        pltpu.VMEM((GX,), jnp.float32),
        pltpu.VMEM((GX,), jnp.float32),
        pltpu.VMEM((GX,), jnp.float32),
        pltpu.VMEM((GX, 16), jnp.float32),
        pltpu.VMEM_SHARED((NP, 16), jnp.float32),
        pltpu.SemaphoreType.DMA,
        pltpu.SemaphoreType.DMA,
    ],
    compiler_params=pltpu.CompilerParams(
        needs_layout_passes=False, use_tc_tiling_on_sc=False),
)(_scatx_body)


def _node_body(nf, cf, part, paux, wn1, bn1, wn2, bn2, gam, bet, h_out, x_out):
    h_neigh = part[0, :N] + part[1, :N]
    aux = paux[0, :N] + paux[1, :N]
    xsum = aux[:, 0:3]
    deg = aux[:, 3:4]
    hin = jnp.concatenate([nf[...], h_neigh], axis=1)
    h1 = _silu(jnp.dot(hin, wn1[...], preferred_element_type=jnp.float32) + bn1[...])
    h2 = jnp.dot(h1, wn2[...], preferred_element_type=jnp.float32) + bn2[...]
    mean = jnp.mean(h2, axis=0, keepdims=True)
    var = jnp.mean(jnp.square(h2 - mean), axis=0, keepdims=True)
    h_out[...] = (h2 - mean) / jnp.sqrt(var + 1e-5) * gam[...] + bet[...]
    x_out[...] = cf[...] + xsum / jnp.maximum(deg, 1.0)


def _node_call(nf, cf, part, paux, wn1, bn1, wn2, bn2, gam, bet):
    return pl.pallas_call(
        _node_body,
        out_shape=(
            jax.ShapeDtypeStruct((N, O_), jnp.float32),
            jax.ShapeDtypeStruct((N, 3), jnp.float32),
        ),
    )(nf, cf, part, paux, wn1, bn1, wn2, bn2, gam, bet)


@jax.jit
def kernel(node_feat, coord_feat, edge_index, edge_feat, W_e1, b_e1, W_e2,
           b_e2, W_n1, b_n1, W_n2, b_n2, W_c1, b_c1, W_c2, bn_gamma, bn_beta):
    src = edge_index[0]
    dst = edge_index[1]
    dst3a = dst.reshape(NW, NGS, NKS, SCH)
    dst3b = dst.reshape(NW, NCH2, SCH2)
    cx = coord_feat[:, 0]
    cy = coord_feat[:, 1]
    cz = coord_feat[:, 2]

    wpq = jnp.concatenate([W_e1[:D], W_e1[D:2 * D]], axis=1)
    bpq = jnp.concatenate([jnp.zeros((H,), jnp.float32), b_e1]).reshape(1, 2 * H)
    wr = W_e1[2 * D:2 * D + 1]
    wef = W_e1[2 * D + 1:]

    efp = jnp.concatenate(
        [edge_feat, jnp.zeros((EP - E, EF), jnp.float32)], axis=0)

    p, q = _pq_call(node_feat, wpq, bpq)
    o12, dxo, dyo, dzo, rado = _gather_call(p, q, cx, cy, cz, src, dst)
    m, s2 = _edge_call(o12, rado.reshape(EP // 128, 128), efp, wr, wef,
                       W_e2, b_e2.reshape(1, H), W_c1, b_c1.reshape(1, H),
                       W_c2.reshape(H))
    part = _scath_call(m, dst3a, jnp.zeros((NP, D), jnp.float32))
    paux = _scatx_call(s2.reshape(EP), dxo, dyo, dzo, dst3b,
                       jnp.zeros((NP, 16), jnp.float32))
    h, x = _node_call(node_feat, coord_feat, part, paux, W_n1,
                      b_n1.reshape(1, H), W_n2, b_n2.reshape(1, O_),
                      bn_gamma.reshape(1, O_), bn_beta.reshape(1, O_))
    return (h, x)
